# Initial kernel scaffold; baseline (speedup 1.0000x reference)
import jax
import jax.numpy as jnp
from jax.experimental import pallas as pl


def kernel(s_t, s_i, gds, params):
    raise NotImplementedError("write your pallas kernel here")


# speedup vs baseline: 3.1431x; 3.1431x over previous
import functools

import jax
import jax.numpy as jnp
from jax.experimental import pallas as pl
from jax.experimental.pallas import tpu as pltpu

D = 256
TEMPERATURE = 10.0
BLOCK = 512
OUT_W = 16


def _prep_kernel(a1, a2, b1, b2, bvi, bot, bvt, boi,
                 gwc, gbc, wgc, gws, gbs, wgs,
                 mt, mi, bt, bi, uc, cc, us, cs):
    f32 = jnp.float32
    mt[:] = jnp.dot(a1[:], a2[:], preferred_element_type=f32)
    mi[:] = jnp.dot(b1[:], b2[:], preferred_element_type=f32)
    bt[:] = jnp.dot(bvi[:], a2[:], preferred_element_type=f32) + bot[:]
    bi[:] = jnp.dot(bvt[:], b2[:], preferred_element_type=f32) + boi[:]
    uc[:] = jnp.dot(gwc[:], wgc[:], preferred_element_type=f32)
    cc[:] = jnp.dot(gbc[:], wgc[:], preferred_element_type=f32)
    us[:] = jnp.dot(gws[:], wgs[:], preferred_element_type=f32)
    cs[:] = jnp.dot(gbs[:], wgs[:], preferred_element_type=f32)


def _fold_weights(p):
    f32 = jnp.float32
    a1 = p['ca_kvpi_w'][D:].T
    a2 = p['ca_opt_w'].T
    b1 = p['ca_kvpt_w'][D:].T
    b2 = p['ca_opi_w'].T
    bvi = p['ca_kvpi_b'][D:][None]
    bot = p['ca_opt_b'][None]
    bvt = p['ca_kvpt_b'][D:][None]
    boi = p['ca_opi_b'][None]
    gwc = p['cb_gds_w'][:, 0][None]
    gbc = p['cb_gds_b'][None]
    wgc = p['cb_c0_w'][:, 2 * D:].T
    gws = p['sh_gds_w'][:, 0][None]
    gbs = p['sh_gds_b'][None]
    wgs = p['sh_h0_w'][:, 2 * D:].T
    shapes = [
        jax.ShapeDtypeStruct((D, D), f32),
        jax.ShapeDtypeStruct((D, D), f32),
        jax.ShapeDtypeStruct((1, D), f32),
        jax.ShapeDtypeStruct((1, D), f32),
        jax.ShapeDtypeStruct((1, D), f32),
        jax.ShapeDtypeStruct((1, D), f32),
        jax.ShapeDtypeStruct((1, 128), f32),
        jax.ShapeDtypeStruct((1, 128), f32),
    ]
    return pl.pallas_call(_prep_kernel, out_shape=shapes)(
        a1, a2, b1, b2, bvi, bot, bvt, boi, gwc, gbc, wgc, gws, gbs, wgs)


def _gelu_exact(x):
    return 0.5 * x * (1.0 + jax.lax.erf(x * 0.7071067811865476))


def _ln(x, g, b, eps=1e-5):
    m = jnp.mean(x, axis=-1, keepdims=True)
    c = x - m
    v = jnp.mean(c * c, axis=-1, keepdims=True)
    return c * jax.lax.rsqrt(v + eps) * g + b


def _main_kernel(xt_ref, xi_ref, g_ref,
                 ct_ref, ci_ref, bt_ref, bi_ref,
                 lntg_ref, lntb_ref, lnig_ref, lnib_ref,
                 wt_ref, wi_ref, u_ref, brow_ref,
                 w1c_ref, b1c_ref, n1_ref, b1n_ref, b0n_ref,
                 whead_ref, bhead_ref, lt_ref,
                 out_ref):
    f32 = jnp.float32
    xt = xt_ref[:]
    xi = xi_ref[:]
    g = g_ref[:]
    pt = jnp.dot(xt, ct_ref[:], preferred_element_type=f32)
    pi = jnp.dot(xi, ci_ref[:], preferred_element_type=f32)
    i_out = pt[:, :D] + bi_ref[:]
    n_t = pt[:, D:]
    t_out = pi[:, :D] + bt_ref[:]
    n_i = pi[:, D:]
    t_refv = _ln(xt + t_out, lntg_ref[:], lntb_ref[:])
    i_refv = _ln(xi + i_out, lnig_ref[:], lnib_ref[:])
    q = (jnp.dot(t_refv, wt_ref[:], preferred_element_type=f32)
         + jnp.dot(i_refv, wi_ref[:], preferred_element_type=f32)
         + g * u_ref[:] + brow_ref[:])
    qa = _gelu_exact(q)
    h0 = qa[:, :D]
    hs = qa[:, D:]
    h1 = _gelu_exact(
        jnp.dot(h0, w1c_ref[:], preferred_element_type=f32) + b1c_ref[:])
    n0 = _gelu_exact(n_t + n_i + b0n_ref[:])
    n1 = _gelu_exact(
        jnp.dot(n0, n1_ref[:], preferred_element_type=f32) + b1n_ref[:])
    hcat = jnp.concatenate([h1, n1, hs], axis=1)
    heads = jnp.dot(hcat, whead_ref[:], preferred_element_type=f32) + bhead_ref[:]
    conflict = heads[:, 0:3]
    normal = heads[:, 3:6]
    sarcasm = heads[:, 6:8]
    tau = jax.nn.sigmoid(lt_ref[:])
    gate = jax.nn.sigmoid((g - tau) * TEMPERATURE)
    logits = gate * conflict + (1.0 - gate) * normal
    routing = (g > tau).astype(f32)
    pad = jnp.zeros_like(heads[:, 0:4])
    out_ref[:] = jnp.concatenate(
        [logits, normal, conflict, sarcasm, routing, pad], axis=1)


@jax.jit
def _run(s_t, s_i, gds, params):
    f32 = jnp.float32
    p = params
    mt, mi, bt, bi, uc, cc, us, cs = _fold_weights(p)
    c_t = jnp.concatenate([mi, p['nb_m0_w'][:, :D].T], axis=1)
    c_i = jnp.concatenate([mt, p['nb_m0_w'][:, D:].T], axis=1)
    w_t = jnp.concatenate([p['cb_c0_w'][:, :D].T, p['sh_h0_w'][:, :D].T], axis=1)
    w_i = jnp.concatenate([p['cb_c0_w'][:, D:2 * D].T, p['sh_h0_w'][:, D:2 * D].T], axis=1)
    u_row = jnp.concatenate([uc, us], axis=1)
    b_row = jnp.concatenate([p['cb_c0_b'][None] + cc, p['sh_h0_b'][None] + cs], axis=1)
    w1c = p['cb_c1_w'].T
    b1c = p['cb_c1_b'][None]
    n1w = p['nb_m1_w'].T
    b1n = p['nb_m1_b'][None]
    b0n = p['nb_m0_b'][None]
    whead = jnp.zeros((384, OUT_W), f32)
    whead = whead.at[0:128, 0:3].set(p['cb_c2_w'].T)
    whead = whead.at[128:256, 3:6].set(p['nb_m2_w'].T)
    whead = whead.at[256:384, 6:8].set(p['sh_h1_w'].T)
    bhead = jnp.zeros((1, OUT_W), f32)
    bhead = bhead.at[0, 0:3].set(p['cb_c2_b'])
    bhead = bhead.at[0, 3:6].set(p['nb_m2_b'])
    bhead = bhead.at[0, 6:8].set(p['sh_h1_b'])
    lt = p['log_threshold'].reshape(1, 1)
    gds2 = gds[:, None]

    B = s_t.shape[0]
    grid = (B // BLOCK,)
    row = lambda i: (i, 0)
    rep = lambda i: (0, 0)
    in_specs = [
        pl.BlockSpec((BLOCK, D), row),
        pl.BlockSpec((BLOCK, D), row),
        pl.BlockSpec((BLOCK, 1), row),
        pl.BlockSpec((D, 2 * D), rep),
        pl.BlockSpec((D, 2 * D), rep),
        pl.BlockSpec((1, D), rep),
        pl.BlockSpec((1, D), rep),
        pl.BlockSpec((1, D), rep),
        pl.BlockSpec((1, D), rep),
        pl.BlockSpec((1, D), rep),
        pl.BlockSpec((1, D), rep),
        pl.BlockSpec((D, 384), rep),
        pl.BlockSpec((D, 384), rep),
        pl.BlockSpec((1, 384), rep),
        pl.BlockSpec((1, 384), rep),
        pl.BlockSpec((D, 128), rep),
        pl.BlockSpec((1, 128), rep),
        pl.BlockSpec((D, 128), rep),
        pl.BlockSpec((1, 128), rep),
        pl.BlockSpec((1, D), rep),
        pl.BlockSpec((384, OUT_W), rep),
        pl.BlockSpec((1, OUT_W), rep),
        pl.BlockSpec((1, 1), rep),
    ]
    packed = pl.pallas_call(
        _main_kernel,
        grid=grid,
        in_specs=in_specs,
        out_specs=pl.BlockSpec((BLOCK, OUT_W), row),
        out_shape=jax.ShapeDtypeStruct((B, OUT_W), f32),
        compiler_params=pltpu.CompilerParams(
            dimension_semantics=("arbitrary",)),
    )(s_t, s_i, gds2, c_t, c_i, bt, bi,
      p['ca_lnt_g'][None], p['ca_lnt_b'][None],
      p['ca_lni_g'][None], p['ca_lni_b'][None],
      w_t, w_i, u_row, b_row, w1c, b1c, n1w, b1n, b0n,
      whead, bhead, lt)
    logits = packed[:, 0:3]
    normal = packed[:, 3:6]
    conflict = packed[:, 6:9]
    sarcasm = packed[:, 9:11]
    routing = packed[:, 11]
    return logits, routing, normal, conflict, sarcasm


def kernel(s_t, s_i, gds, params):
    return _run(s_t, s_i, gds, params)
